# baseline (speedup 1.0000x reference)
import functools

import jax
import jax.numpy as jnp
from jax.experimental import pallas as pl


def _mm_body(a_ref, w_ref, b_ref, o_ref, *, act, chunk, desc):
    a = a_ref[...].reshape(a_ref.shape[-2], a_ref.shape[-1])
    w = w_ref[...].reshape(w_ref.shape[-2], w_ref.shape[-1])
    Kd = a.shape[1]
    ch = chunk if chunk else Kd
    parts = [jnp.dot(a[:, lo:lo + ch], w[lo:lo + ch, :],
                     preferred_element_type=jnp.float32)
             for lo in range(0, Kd, ch)]
    if desc:
        parts = parts[::-1]
    y = parts[0]
    for p in parts[1:]:
        y = y + p
    y = y + b_ref[...].reshape(1, b_ref.shape[-1])
    if act == "relu":
        y = jnp.maximum(y, 0.0)
    elif act == "tanh":
        y = jnp.tanh(y)
    o_ref[...] = y.reshape(o_ref.shape)


def _mm(A, W, b, act="none", block_m=2048, chunk=0, desc=False):
    M, K = A.shape
    N = W.shape[1]
    grid = (M // block_m,)
    return pl.pallas_call(
        functools.partial(_mm_body, act=act, chunk=chunk, desc=desc),
        grid=grid,
        in_specs=[
            pl.BlockSpec((block_m, K), lambda i: (i, 0)),
            pl.BlockSpec((K, N), lambda i: (0, 0)),
            pl.BlockSpec((1, N), lambda i: (0, 0)),
        ],
        out_specs=pl.BlockSpec((block_m, N), lambda i: (i, 0)),
        out_shape=jax.ShapeDtypeStruct((M, N), jnp.float32),
    )(A, W, b.reshape(1, N))


def _mm4(A, W, b, act="none", block_m=2048):
    _, M, K = A.shape
    N = W.shape[2]
    grid = (4, M // block_m)
    return pl.pallas_call(
        functools.partial(_mm_body, act=act, chunk=0, desc=False),
        grid=grid,
        in_specs=[
            pl.BlockSpec((1, block_m, K), lambda p, i: (p, i, 0)),
            pl.BlockSpec((1, K, N), lambda p, i: (p, 0, 0)),
            pl.BlockSpec((1, 1, N), lambda p, i: (0, 0, 0)),
        ],
        out_specs=pl.BlockSpec((1, block_m, N), lambda p, i: (p, i, 0)),
        out_shape=jax.ShapeDtypeStruct((4, M, N), jnp.float32),
    )(A, W, b.reshape(1, 1, N))


def _vq_body(z_ref, cb_ref, xn_ref, yn_ref, idx_ref, q_ref, pool_ref, loss_ref):
    z = z_ref[...]
    cb = cb_ref[...]
    bm = z.shape[0]
    v = cb.shape[0]
    scores = jax.lax.dot_general(
        z, cb, dimension_numbers=(((1,), (1,)), ((), ())),
        preferred_element_type=jnp.float32)
    dist = xn_ref[...] + yn_ref[...] - 2.0 * scores
    dist = jnp.sqrt(jnp.maximum(dist, 0.0))
    m = jnp.min(dist, axis=1, keepdims=True)
    iota = jax.lax.broadcasted_iota(jnp.int32, (bm, v), 1)
    idx = jnp.min(jnp.where(dist <= m, iota, jnp.int32(2**30)), axis=1)
    idx_ref[...] = idx[None, None, :]
    onehot = (iota == idx[:, None]).astype(jnp.float32)
    q = jnp.dot(onehot, cb, preferred_element_type=jnp.float32)
    q_ref[...] = q
    pool_ref[...] = jnp.mean(q.reshape(bm // 64, 64, 64), axis=1)
    d = q - z
    loss_ref[...] = jnp.sum(d * d).reshape(1, 1, 1)


def _vq(z_flat, codebook, x_norm, y_norm, block_m=2048):
    M, ed = z_flat.shape
    V = codebook.shape[0]
    nb = M // block_m
    idx, q, pool, loss = pl.pallas_call(
        _vq_body,
        grid=(nb,),
        in_specs=[
            pl.BlockSpec((block_m, ed), lambda i: (i, 0)),
            pl.BlockSpec((V, ed), lambda i: (0, 0)),
            pl.BlockSpec((block_m, 1), lambda i: (i, 0)),
            pl.BlockSpec((1, V), lambda i: (0, 0)),
        ],
        out_specs=[
            pl.BlockSpec((1, 1, block_m), lambda i: (i, 0, 0)),
            pl.BlockSpec((block_m, ed), lambda i: (i, 0)),
            pl.BlockSpec((block_m // 64, ed), lambda i: (i, 0)),
            pl.BlockSpec((1, 1, 1), lambda i: (i, 0, 0)),
        ],
        out_shape=[
            jax.ShapeDtypeStruct((nb, 1, block_m), jnp.int32),
            jax.ShapeDtypeStruct((M, ed), jnp.float32),
            jax.ShapeDtypeStruct((M // 64, ed), jnp.float32),
            jax.ShapeDtypeStruct((nb, 1, 1), jnp.float32),
        ],
    )(z_flat, codebook, x_norm, y_norm)
    return idx.reshape(M), q, pool, loss


def _enc_patches(x_nhwc, cgroup=0):
    N, H, W, C = x_nhwc.shape
    OH, OW = H // 2, W // 2
    xp = jnp.pad(x_nhwc, ((0, 0), (1, 1), (1, 1), (0, 0)))
    taps = []
    for kh in range(4):
        for kw in range(4):
            taps.append(xp[:, kh::2, kw::2, :][:, :OH, :OW, :])
    A = jnp.stack(taps, axis=-2)
    if cgroup:
        g = cgroup
        A = A.reshape(N, OH, OW, 16, C // g, g)
        A = jnp.transpose(A, (0, 1, 2, 4, 3, 5))
    return A.reshape(N * OH * OW, 16 * C)


def _enc_w(w, cgroup=0):
    O, C = w.shape[0], w.shape[1]
    wm = jnp.transpose(w, (2, 3, 1, 0))
    if cgroup:
        g = cgroup
        wm = wm.reshape(16, C // g, g, O)
        wm = jnp.transpose(wm, (1, 0, 2, 3))
    return wm.reshape(16 * C, O)


def _dec_patches(x_nhwc):
    N, H, W, C = x_nhwc.shape
    xp = jnp.pad(x_nhwc, ((0, 0), (1, 1), (1, 1), (0, 0)))
    pars = []
    for pi in (0, 1):
        for pj in (0, 1):
            taps = []
            for th in (0, 1):
                for tw in (0, 1):
                    taps.append(xp[:, pi + th:pi + th + H,
                                   pj + tw:pj + tw + W, :])
            pars.append(jnp.stack(taps, axis=-2).reshape(N * H * W, 4 * C))
    return jnp.stack(pars, axis=0)


def _dec_w(w):
    mats = []
    for pi in (0, 1):
        for pj in (0, 1):
            sub = w[:, :, pi::2, pj::2]
            mats.append(jnp.transpose(sub, (2, 3, 1, 0)).reshape(-1, w.shape[0]))
    return jnp.stack(mats, axis=0)


def _interleave(y4, N, H, W, O):
    y = y4.reshape(2, 2, N, H, W, O)
    y = jnp.transpose(y, (2, 3, 0, 4, 1, 5))
    return y.reshape(N, 2 * H, 2 * W, O)


def kernel(x, conv1_w, conv1_b, conv2_w, conv2_b, conv3_w, conv3_b, codebook,
           dec1_w, dec1_b, dec2_w, dec2_b, dec3_w, dec3_b):
    B, T, H, W = x.shape
    NF = B * T
    V, ed = codebook.shape

    x0 = x.reshape(NF, H, W, 1)
    h1 = _mm(_enc_patches(x0), _enc_w(conv1_w), conv1_b, act="relu",
             block_m=8192)
    h1 = h1.reshape(NF, H // 2, W // 2, 32)
    h2 = _mm(_enc_patches(h1, cgroup=8), _enc_w(conv2_w, cgroup=8), conv2_b,
             act="relu", block_m=4096, chunk=128, desc=True)
    h2 = h2.reshape(NF, H // 4, W // 4, 64)
    z_flat = _mm(_enc_patches(h2), _enc_w(conv3_w), conv3_b, act="none",
                 block_m=2048, chunk=256, desc=False)

    x_norm = jnp.sum(z_flat ** 2, axis=1, keepdims=True)
    y_norm = jnp.sum(codebook ** 2, axis=1)[None, :]
    indices, quantized, pooled, loss_parts = _vq(z_flat, codebook,
                                                 x_norm, y_norm)
    vq_loss = 1.25 * jnp.sum(loss_parts) / (z_flat.size)
    quantized_pooled = pooled.reshape(B, T, ed)
    indices_seq = indices.reshape(B, T, 64)[:, :, 0]

    q_img = quantized.reshape(NF, 8, 8, ed)
    d1 = _mm4(_dec_patches(q_img), _dec_w(dec1_w), dec1_b, act="relu",
              block_m=2048)
    d1 = _interleave(d1, NF, 8, 8, 64)
    d2 = _mm4(_dec_patches(d1), _dec_w(dec2_w), dec2_b, act="relu",
              block_m=4096)
    d2 = _interleave(d2, NF, 16, 16, 32)
    d3 = _mm4(_dec_patches(d2), _dec_w(dec3_w), dec3_b, act="tanh",
              block_m=8192)
    recon = _interleave(d3, NF, 32, 32, 1)
    reconstructed = jnp.transpose(recon, (0, 3, 1, 2))

    return (quantized_pooled, indices_seq, vq_loss, reconstructed)
